# initial kernel scaffold (unmeasured)
import jax
import jax.numpy as jnp
from jax import lax
from jax.experimental import pallas as pl
from jax.experimental.pallas import tpu as pltpu

N_Z = 4
SCALE = 128 ** -0.5


def _partial_body(q_ref, k_ref, v_ref, u_ref, l_ref):
    q = q_ref[0, :, 0, :]
    k = k_ref[0, :, 0, :]
    v = v_ref[0, :, 0, :]
    s = lax.dot_general(
        q, k, (((1,), (1,)), ((), ())), preferred_element_type=jnp.float32
    )
    p = jnp.exp(s * SCALE)
    l = jnp.sum(p, axis=1, keepdims=True)
    u = lax.dot_general(
        p, v, (((1,), (0,)), ((), ())), preferred_element_type=jnp.float32
    )
    u_ref[0, :, 0, :] = u
    l_ref[0, :, 0, :] = jnp.broadcast_to(l, u.shape)


def _allreduce_body(u_ref, l_ref, o_ref, accl_ref, comm_ref, send_sems, recv_sems):
    my_x = lax.axis_index("x")
    my_y = lax.axis_index("y")
    my_z = lax.axis_index("z")
    left = (my_z - 1) % N_Z
    right = (my_z + 1) % N_Z

    barrier_sem = pltpu.get_barrier_semaphore()
    for nbr in (left, right):
        pl.semaphore_signal(
            barrier_sem,
            inc=1,
            device_id=(my_x, my_y, nbr),
            device_id_type=pl.DeviceIdType.MESH,
        )
    pl.semaphore_wait(barrier_sem, 2)

    o_ref[...] = u_ref[...]
    accl_ref[...] = l_ref[...]
    comm_ref[0, 0] = u_ref[...]
    comm_ref[0, 1] = l_ref[...]

    for h in range(N_Z - 1):
        rdma = pltpu.make_async_remote_copy(
            src_ref=comm_ref.at[h],
            dst_ref=comm_ref.at[h + 1],
            send_sem=send_sems.at[h],
            recv_sem=recv_sems.at[h + 1],
            device_id=(my_x, my_y, right),
            device_id_type=pl.DeviceIdType.MESH,
        )
        rdma.start()
        rdma.wait()
        o_ref[...] += comm_ref[h + 1, 0]
        accl_ref[...] += comm_ref[h + 1, 1]

    o_ref[...] = o_ref[...] / accl_ref[...]


def kernel(Q, K, V):
    b, sq, h, d = Q.shape
    skv = K.shape[1]

    u, l = pl.pallas_call(
        _partial_body,
        grid=(b, h),
        in_specs=[
            pl.BlockSpec((1, sq, 1, d), lambda i, j: (i, 0, j, 0)),
            pl.BlockSpec((1, skv, 1, d), lambda i, j: (i, 0, j, 0)),
            pl.BlockSpec((1, skv, 1, d), lambda i, j: (i, 0, j, 0)),
        ],
        out_specs=[
            pl.BlockSpec((1, sq, 1, d), lambda i, j: (i, 0, j, 0)),
            pl.BlockSpec((1, sq, 1, d), lambda i, j: (i, 0, j, 0)),
        ],
        out_shape=[
            jax.ShapeDtypeStruct((b, sq, h, d), jnp.float32),
            jax.ShapeDtypeStruct((b, sq, h, d), jnp.float32),
        ],
    )(Q, K, V)

    return pl.pallas_call(
        _allreduce_body,
        in_specs=[
            pl.BlockSpec(memory_space=pltpu.VMEM),
            pl.BlockSpec(memory_space=pltpu.VMEM),
        ],
        out_specs=pl.BlockSpec(memory_space=pltpu.VMEM),
        out_shape=jax.ShapeDtypeStruct((b, sq, h, d), jnp.float32),
        scratch_shapes=[
            pltpu.VMEM((b, sq, h, d), jnp.float32),
            pltpu.VMEM((N_Z, 2, b, sq, h, d), jnp.float32),
            pltpu.SemaphoreType.DMA((N_Z,)),
            pltpu.SemaphoreType.DMA((N_Z,)),
        ],
        compiler_params=pltpu.CompilerParams(collective_id=0),
    )(u, l)


# baseline (device time: 246543 ns/iter reference)
import jax
import jax.numpy as jnp
from jax import lax
from jax.experimental import pallas as pl
from jax.experimental.pallas import tpu as pltpu

N_Z = 4
H = 8
D = 128
SCALE = D ** -0.5
KV_CHUNK = 512


def _partial_body(q_ref, k_ref, v_ref, u_ref, l_ref):
    kvc = pl.program_id(1)

    @pl.when(kvc == 0)
    def _():
        u_ref[...] = jnp.zeros_like(u_ref)
        l_ref[...] = jnp.zeros_like(l_ref)

    for hh in range(H):
        sl = pl.ds(hh * D, D)
        q = q_ref[0, :, sl]
        k = k_ref[0, :, sl]
        v = v_ref[0, :, sl]
        s = lax.dot_general(
            q, k, (((1,), (1,)), ((), ())), preferred_element_type=jnp.float32
        )
        p = jnp.exp(s * SCALE)
        l = jnp.sum(p, axis=1, keepdims=True)
        u = lax.dot_general(
            p, v, (((1,), (0,)), ((), ())), preferred_element_type=jnp.float32
        )
        u_ref[0, :, sl] += u
        l_ref[0, :, sl] += jnp.broadcast_to(l, u.shape)


def _allreduce_body(u_ref, l_ref, o_ref, accl_ref, comm_ref, send_sems, recv_sems):
    my_x = lax.axis_index("x")
    my_y = lax.axis_index("y")
    my_z = lax.axis_index("z")
    left = (my_z - 1) % N_Z
    right = (my_z + 1) % N_Z

    barrier_sem = pltpu.get_barrier_semaphore()
    for nbr in (left, right):
        pl.semaphore_signal(
            barrier_sem,
            inc=1,
            device_id=(my_x, my_y, nbr),
            device_id_type=pl.DeviceIdType.MESH,
        )
    pl.semaphore_wait(barrier_sem, 2)

    o_ref[...] = u_ref[...]
    accl_ref[...] = l_ref[...]
    comm_ref[0, 0] = u_ref[...]
    comm_ref[0, 1] = l_ref[...]

    for h in range(N_Z - 1):
        rdma = pltpu.make_async_remote_copy(
            src_ref=comm_ref.at[h],
            dst_ref=comm_ref.at[h + 1],
            send_sem=send_sems.at[h],
            recv_sem=recv_sems.at[h + 1],
            device_id=(my_x, my_y, right),
            device_id_type=pl.DeviceIdType.MESH,
        )
        rdma.start()
        rdma.wait()
        o_ref[...] += comm_ref[h + 1, 0]
        accl_ref[...] += comm_ref[h + 1, 1]

    o_ref[...] = o_ref[...] / accl_ref[...]


def kernel(Q, K, V):
    b, sq, h, d = Q.shape
    skv = K.shape[1]
    hd = h * d
    n_chunks = skv // KV_CHUNK

    Q2 = Q.reshape(b, sq, hd)
    K2 = K.reshape(b, skv, hd)
    V2 = V.reshape(b, skv, hd)

    u, l = pl.pallas_call(
        _partial_body,
        grid=(b, n_chunks),
        in_specs=[
            pl.BlockSpec((1, sq, hd), lambda i, c: (i, 0, 0)),
            pl.BlockSpec((1, KV_CHUNK, hd), lambda i, c: (i, c, 0)),
            pl.BlockSpec((1, KV_CHUNK, hd), lambda i, c: (i, c, 0)),
        ],
        out_specs=[
            pl.BlockSpec((1, sq, hd), lambda i, c: (i, 0, 0)),
            pl.BlockSpec((1, sq, hd), lambda i, c: (i, 0, 0)),
        ],
        out_shape=[
            jax.ShapeDtypeStruct((b, sq, hd), jnp.float32),
            jax.ShapeDtypeStruct((b, sq, hd), jnp.float32),
        ],
    )(Q2, K2, V2)

    o = pl.pallas_call(
        _allreduce_body,
        in_specs=[
            pl.BlockSpec(memory_space=pltpu.VMEM),
            pl.BlockSpec(memory_space=pltpu.VMEM),
        ],
        out_specs=pl.BlockSpec(memory_space=pltpu.VMEM),
        out_shape=jax.ShapeDtypeStruct((b, sq, hd), jnp.float32),
        scratch_shapes=[
            pltpu.VMEM((b, sq, hd), jnp.float32),
            pltpu.VMEM((N_Z, 2, b, sq, hd), jnp.float32),
            pltpu.SemaphoreType.DMA((N_Z,)),
            pltpu.SemaphoreType.DMA((N_Z,)),
        ],
        compiler_params=pltpu.CompilerParams(collective_id=0),
    )(u, l)

    return o.reshape(b, sq, h, d)


# device time: 133991 ns/iter; 1.8400x vs baseline; 1.8400x over previous
import jax
import jax.numpy as jnp
from jax import lax
from jax.experimental import pallas as pl
from jax.experimental.pallas import tpu as pltpu

N_Z = 4
H = 8
D = 128
SCALE = D ** -0.5
KV_CHUNK = 512


def _partial_body(q_ref, k_ref, v_ref, u_ref, l_ref):
    kvc = pl.program_id(1)

    @pl.when(kvc == 0)
    def _():
        u_ref[...] = jnp.zeros_like(u_ref)
        l_ref[...] = jnp.zeros_like(l_ref)

    for hh in range(H):
        q = q_ref[0, :, hh, :]
        k = k_ref[0, :, hh, :]
        v = v_ref[0, :, hh, :]
        s = lax.dot_general(
            q, k, (((1,), (1,)), ((), ())), preferred_element_type=jnp.float32
        )
        p = jnp.exp(s * SCALE)
        l = jnp.sum(p, axis=1, keepdims=True)
        u = lax.dot_general(
            p, v, (((1,), (0,)), ((), ())), preferred_element_type=jnp.float32
        )
        u_ref[0, :, hh, :] += u
        l_ref[0, :, hh, :] += jnp.broadcast_to(l, u.shape)


def _allreduce_body(u_ref, l_ref, o_ref, accl_ref, comm_ref, send_sems, recv_sems):
    my_x = lax.axis_index("x")
    my_y = lax.axis_index("y")
    my_z = lax.axis_index("z")
    left = (my_z - 1) % N_Z
    right = (my_z + 1) % N_Z

    barrier_sem = pltpu.get_barrier_semaphore()
    for nbr in (left, right):
        pl.semaphore_signal(
            barrier_sem,
            inc=1,
            device_id=(my_x, my_y, nbr),
            device_id_type=pl.DeviceIdType.MESH,
        )
    pl.semaphore_wait(barrier_sem, 2)

    o_ref[...] = u_ref[...]
    accl_ref[...] = l_ref[...]
    comm_ref[0, 0] = u_ref[...]
    comm_ref[0, 1] = l_ref[...]

    for h in range(N_Z - 1):
        rdma = pltpu.make_async_remote_copy(
            src_ref=comm_ref.at[h],
            dst_ref=comm_ref.at[h + 1],
            send_sem=send_sems.at[h],
            recv_sem=recv_sems.at[h + 1],
            device_id=(my_x, my_y, right),
            device_id_type=pl.DeviceIdType.MESH,
        )
        rdma.start()
        rdma.wait()
        o_ref[...] += comm_ref[h + 1, 0]
        accl_ref[...] += comm_ref[h + 1, 1]

    o_ref[...] = o_ref[...] / accl_ref[...]


def partial_only(Q, K, V):
    b, sq, h, d = Q.shape
    skv = K.shape[1]
    n_chunks = skv // KV_CHUNK

    return pl.pallas_call(
        _partial_body,
        grid=(b, n_chunks),
        in_specs=[
            pl.BlockSpec((1, sq, h, d), lambda i, c: (i, 0, 0, 0)),
            pl.BlockSpec((1, KV_CHUNK, h, d), lambda i, c: (i, c, 0, 0)),
            pl.BlockSpec((1, KV_CHUNK, h, d), lambda i, c: (i, c, 0, 0)),
        ],
        out_specs=[
            pl.BlockSpec((1, sq, h, d), lambda i, c: (i, 0, 0, 0)),
            pl.BlockSpec((1, sq, h, d), lambda i, c: (i, 0, 0, 0)),
        ],
        out_shape=[
            jax.ShapeDtypeStruct((b, sq, h, d), jnp.float32),
            jax.ShapeDtypeStruct((b, sq, h, d), jnp.float32),
        ],
    )(Q, K, V)


def kernel(Q, K, V):
    b, sq, h, d = Q.shape
    u, l = partial_only(Q, K, V)

    return pl.pallas_call(
        _allreduce_body,
        in_specs=[
            pl.BlockSpec(memory_space=pltpu.VMEM),
            pl.BlockSpec(memory_space=pltpu.VMEM),
        ],
        out_specs=pl.BlockSpec(memory_space=pltpu.VMEM),
        out_shape=jax.ShapeDtypeStruct((b, sq, h, d), jnp.float32),
        scratch_shapes=[
            pltpu.VMEM((b, sq, h, d), jnp.float32),
            pltpu.VMEM((N_Z, 2, b, sq, h, d), jnp.float32),
            pltpu.SemaphoreType.DMA((N_Z,)),
            pltpu.SemaphoreType.DMA((N_Z,)),
        ],
        compiler_params=pltpu.CompilerParams(collective_id=0),
    )(u, l)
